# baseline (speedup 1.0000x reference)
import functools

import jax
import jax.numpy as jnp
from jax import lax
from jax.experimental import pallas as pl
from jax.experimental.pallas import tpu as pltpu
from jax.experimental.pallas import tpu_sc as plsc

N = 100000
D = 128
G = 512
L = 16

C = 128
FULL_CHUNKS = N // C
TAIL = N - FULL_CHUNKS * C
NW = 32
NBUF = 3
CPW = 25
ROUNDS = (CPW + NBUF - 1) // NBUF
ROWS_PER_TILE = G // 16

_mesh = plsc.VectorSubcoreMesh(core_axis_name="c", subcore_axis_name="s")

_scratch = (
    [pltpu.VMEM((C, D), jnp.float32) for _ in range(NBUF)]
    + [pltpu.VMEM((C,), jnp.int32) for _ in range(NBUF)]
    + [pltpu.VMEM((TAIL,), jnp.int32),
       pltpu.VMEM((TAIL, D), jnp.float32),
       pltpu.VMEM((G, D), jnp.float32),
       pltpu.VMEM((C,), jnp.int32),
       pltpu.VMEM_SHARED((G, D), jnp.float32)]
    + [pltpu.SemaphoreType.DMA for _ in range(2 * NBUF + 1)]
)


@functools.partial(
    pl.kernel,
    out_type=jax.ShapeDtypeStruct((2, G, D), jnp.float32),
    mesh=_mesh,
    scratch_types=_scratch,
)
def _sc_segment_sum(x_hbm, b_hbm, out_hbm, *refs):
    r_v = refs[0:NBUF]
    i_v = refs[NBUF:2 * NBUF]
    tidx_v, trows_v, lacc_v, ident_v, acc_sh = refs[2 * NBUF:2 * NBUF + 5]
    rsem = refs[2 * NBUF + 5:2 * NBUF + 5 + NBUF]
    isem = refs[2 * NBUF + 5 + NBUF:2 * NBUF + 5 + 2 * NBUF]
    dsem = refs[2 * NBUF + 5 + 2 * NBUF]

    cid = lax.axis_index("c")
    sid = lax.axis_index("s")
    wid = cid * 16 + sid
    g0 = wid * CPW

    def valid(c):
        return g0 + c < FULL_CHUNKS

    def load(c, b):
        if c >= CPW:
            return

        @pl.when(valid(c))
        def _():
            base = (g0 + c) * C
            pltpu.async_copy(b_hbm.at[pl.ds(base, C)], i_v[b], isem[b])
            pltpu.async_copy(x_hbm.at[pl.ds(base, C)], r_v[b], rsem[b])

    lane = lax.iota(jnp.int32, L)
    zv = jnp.zeros((L,), jnp.float32)

    def reduce_rows(rows_ref, ids_ref, nrows):
        def rows_body(ru, carry):
            idv = ids_ref[pl.ds(ru * L, L)]
            for u in range(L):
                r = ru * L + u
                rid = idv[u]
                for k in range(D // L):
                    v = rows_ref[r, pl.ds(k * L, L)]
                    plsc.addupdate(lacc_v.at[rid, pl.ds(k * L, L)], v)
            return carry

        lax.fori_loop(0, nrows // L, rows_body, 0)

    def process(c, b):
        if c >= CPW:
            return

        @pl.when(valid(c))
        def _():
            base = (g0 + c) * C
            pltpu.make_async_copy(b_hbm.at[pl.ds(base, C)], i_v[b],
                                  isem[b]).wait()
            pltpu.make_async_copy(x_hbm.at[pl.ds(base, C)], r_v[b],
                                  rsem[b]).wait()
            reduce_rows(r_v[b], i_v[b], C)

    for b in range(NBUF):
        load(b, b)

    def zero_body(j, carry):
        for k in range(D // L):
            lacc_v[j, pl.ds(k * L, L)] = zv
        return carry

    lax.fori_loop(0, G, zero_body, 0)
    for k in range(C // L):
        ident_v[pl.ds(k * L, L)] = lane + (k * L)
    pltpu.sync_copy(lacc_v.at[pl.ds(0, ROWS_PER_TILE)],
                    acc_sh.at[pl.ds(sid * ROWS_PER_TILE, ROWS_PER_TILE)])
    plsc.subcore_barrier()

    def dyn_load(c, b):
        @pl.when((c < CPW) & valid(c))
        def _():
            base = (g0 + c) * C
            pltpu.async_copy(b_hbm.at[pl.ds(base, C)], i_v[b], isem[b])
            pltpu.async_copy(x_hbm.at[pl.ds(base, C)], r_v[b], rsem[b])

    def round_body(r, carry):
        for b in range(NBUF):
            c = NBUF * r + b

            @pl.when((c < CPW) & valid(c))
            def _():
                base = (g0 + c) * C
                pltpu.make_async_copy(b_hbm.at[pl.ds(base, C)], i_v[b],
                                      isem[b]).wait()
                pltpu.make_async_copy(x_hbm.at[pl.ds(base, C)], r_v[b],
                                      rsem[b]).wait()
                reduce_rows(r_v[b], i_v[b], C)

            dyn_load(c + NBUF, b)
        return carry

    lax.fori_loop(0, ROUNDS, round_body, 0)

    @pl.when(wid == NW - 1)
    def _():
        tbase = FULL_CHUNKS * C
        pltpu.sync_copy(b_hbm.at[pl.ds(tbase, TAIL)], tidx_v)
        pltpu.sync_copy(x_hbm.at[pl.ds(tbase, TAIL)], trows_v)
        reduce_rows(trows_v, tidx_v, TAIL)

    for q in range(G // C):
        pltpu.async_copy(
            lacc_v.at[pl.ds(q * C, C)],
            acc_sh.at[pl.ds(q * C, C)].at[ident_v], dsem, add=True)
    for q in range(G // C):
        pltpu.make_async_copy(
            lacc_v.at[pl.ds(q * C, C)],
            acc_sh.at[pl.ds(q * C, C)].at[ident_v], dsem).wait()

    plsc.subcore_barrier()

    pltpu.sync_copy(
        acc_sh.at[pl.ds(sid * ROWS_PER_TILE, ROWS_PER_TILE)],
        out_hbm.at[cid, pl.ds(sid * ROWS_PER_TILE, ROWS_PER_TILE)])


def _combine_body(p_ref, o_ref):
    o_ref[...] = p_ref[0] + p_ref[1]


_combine = pl.pallas_call(
    _combine_body,
    out_shape=jax.ShapeDtypeStruct((G, D), jnp.float32),
)


def kernel(input, batch, num_graphs):
    partials = _sc_segment_sum(input, batch.astype(jnp.int32))
    return _combine(partials)


# speedup vs baseline: 1.5291x; 1.5291x over previous
import functools

import jax
import jax.numpy as jnp
from jax import lax
from jax.experimental import pallas as pl
from jax.experimental.pallas import tpu as pltpu
from jax.experimental.pallas import tpu_sc as plsc

N = 100000
D = 128
G = 512
L = 16

C = 128
FULL_CHUNKS = N // C
TAIL = N - FULL_CHUNKS * C
NW = 32
NBUF = 3
CPW = 25
ROUNDS = (CPW + NBUF - 1) // NBUF
ROWS_PER_TILE = G // 16

_mesh = plsc.VectorSubcoreMesh(core_axis_name="c", subcore_axis_name="s")

_scratch = (
    [pltpu.VMEM((C, D), jnp.float32) for _ in range(NBUF)]
    + [pltpu.VMEM((C,), jnp.int32) for _ in range(NBUF)]
    + [pltpu.VMEM((TAIL,), jnp.int32),
       pltpu.VMEM((TAIL, D), jnp.float32),
       pltpu.VMEM((G, D), jnp.float32),
       pltpu.VMEM((C,), jnp.int32),
       pltpu.VMEM_SHARED((G, D), jnp.float32)]
    + [pltpu.SemaphoreType.DMA for _ in range(2 * NBUF + 1)]
)


@functools.partial(
    pl.kernel,
    out_type=jax.ShapeDtypeStruct((2, G, D), jnp.float32),
    mesh=_mesh,
    scratch_types=_scratch,
)
def _sc_segment_sum(x_hbm, b_hbm, out_hbm, *refs):
    r_v = refs[0:NBUF]
    i_v = refs[NBUF:2 * NBUF]
    tidx_v, trows_v, lacc_v, ident_v, acc_sh = refs[2 * NBUF:2 * NBUF + 5]
    rsem = refs[2 * NBUF + 5:2 * NBUF + 5 + NBUF]
    isem = refs[2 * NBUF + 5 + NBUF:2 * NBUF + 5 + 2 * NBUF]
    dsem = refs[2 * NBUF + 5 + 2 * NBUF]

    cid = lax.axis_index("c")
    sid = lax.axis_index("s")
    wid = cid * 16 + sid
    g0 = wid * CPW

    def valid(c):
        return g0 + c < FULL_CHUNKS

    def load(c, b):
        if c >= CPW:
            return

        @pl.when(valid(c))
        def _():
            base = (g0 + c) * C
            pltpu.async_copy(b_hbm.at[pl.ds(base, C)], i_v[b], isem[b])
            pltpu.async_copy(x_hbm.at[pl.ds(base, C)], r_v[b], rsem[b])

    lane = lax.iota(jnp.int32, L)
    zv = jnp.zeros((L,), jnp.float32)

    def reduce_rows(rows_ref, ids_ref, nrows):
        def group(gi, carry):
            idv = ids_ref[pl.ds(gi * L, L)]
            first = idv[0]
            last = idv[L - 1]

            @pl.when(first == last)
            def _():
                for k in range(D // L):
                    a = rows_ref[gi * L, pl.ds(k * L, L)]
                    for u in range(1, L):
                        a = a + rows_ref[gi * L + u, pl.ds(k * L, L)]
                    plsc.addupdate(lacc_v.at[first, pl.ds(k * L, L)], a)

            @pl.when(first != last)
            def _():
                for u in range(L):
                    rid = idv[u]
                    for k in range(D // L):
                        v = rows_ref[gi * L + u, pl.ds(k * L, L)]
                        plsc.addupdate(lacc_v.at[rid, pl.ds(k * L, L)], v)

            return carry

        lax.fori_loop(0, nrows // L, group, 0)

    def process(c, b):
        if c >= CPW:
            return

        @pl.when(valid(c))
        def _():
            base = (g0 + c) * C
            pltpu.make_async_copy(b_hbm.at[pl.ds(base, C)], i_v[b],
                                  isem[b]).wait()
            pltpu.make_async_copy(x_hbm.at[pl.ds(base, C)], r_v[b],
                                  rsem[b]).wait()
            reduce_rows(r_v[b], i_v[b], C)

    for b in range(NBUF):
        load(b, b)

    def zero_body(j, carry):
        for k in range(D // L):
            lacc_v[j, pl.ds(k * L, L)] = zv
        return carry

    lax.fori_loop(0, G, zero_body, 0)
    for k in range(C // L):
        ident_v[pl.ds(k * L, L)] = lane + (k * L)
    pltpu.sync_copy(lacc_v.at[pl.ds(0, ROWS_PER_TILE)],
                    acc_sh.at[pl.ds(sid * ROWS_PER_TILE, ROWS_PER_TILE)])
    plsc.subcore_barrier()

    def dyn_load(c, b):
        @pl.when((c < CPW) & valid(c))
        def _():
            base = (g0 + c) * C
            pltpu.async_copy(b_hbm.at[pl.ds(base, C)], i_v[b], isem[b])
            pltpu.async_copy(x_hbm.at[pl.ds(base, C)], r_v[b], rsem[b])

    def round_body(r, carry):
        for b in range(NBUF):
            c = NBUF * r + b

            @pl.when((c < CPW) & valid(c))
            def _():
                base = (g0 + c) * C
                pltpu.make_async_copy(b_hbm.at[pl.ds(base, C)], i_v[b],
                                      isem[b]).wait()
                pltpu.make_async_copy(x_hbm.at[pl.ds(base, C)], r_v[b],
                                      rsem[b]).wait()
                reduce_rows(r_v[b], i_v[b], C)

            dyn_load(c + NBUF, b)
        return carry

    lax.fori_loop(0, ROUNDS, round_body, 0)

    @pl.when(wid == NW - 1)
    def _():
        tbase = FULL_CHUNKS * C
        pltpu.sync_copy(b_hbm.at[pl.ds(tbase, TAIL)], tidx_v)
        pltpu.sync_copy(x_hbm.at[pl.ds(tbase, TAIL)], trows_v)
        reduce_rows(trows_v, tidx_v, TAIL)

    for q in range(G // C):
        pltpu.async_copy(
            lacc_v.at[pl.ds(q * C, C)],
            acc_sh.at[pl.ds(q * C, C)].at[ident_v], dsem, add=True)
    for q in range(G // C):
        pltpu.make_async_copy(
            lacc_v.at[pl.ds(q * C, C)],
            acc_sh.at[pl.ds(q * C, C)].at[ident_v], dsem).wait()

    plsc.subcore_barrier()

    pltpu.sync_copy(
        acc_sh.at[pl.ds(sid * ROWS_PER_TILE, ROWS_PER_TILE)],
        out_hbm.at[cid, pl.ds(sid * ROWS_PER_TILE, ROWS_PER_TILE)])


def _combine_body(p_ref, o_ref):
    o_ref[...] = p_ref[0] + p_ref[1]


_combine = pl.pallas_call(
    _combine_body,
    out_shape=jax.ShapeDtypeStruct((G, D), jnp.float32),
)


def kernel(input, batch, num_graphs):
    partials = _sc_segment_sum(input, batch.astype(jnp.int32))
    return _combine(partials)


# speedup vs baseline: 2.0677x; 1.3522x over previous
import functools

import jax
import jax.numpy as jnp
from jax import lax
from jax.experimental import pallas as pl
from jax.experimental.pallas import tpu as pltpu
from jax.experimental.pallas import tpu_sc as plsc

N = 100000
D = 128
G = 512
L = 16

C = 128
FULL_CHUNKS = N // C
TAIL = N - FULL_CHUNKS * C
NW = 32
NBUF = 3
CPW = 25
ROUNDS = (CPW + NBUF - 1) // NBUF
ROWS_PER_TILE = G // 16

_mesh = plsc.VectorSubcoreMesh(core_axis_name="c", subcore_axis_name="s")

_scratch = (
    [pltpu.VMEM((C, D), jnp.float32) for _ in range(NBUF)]
    + [pltpu.VMEM((C,), jnp.int32) for _ in range(NBUF)]
    + [pltpu.VMEM((TAIL,), jnp.int32),
       pltpu.VMEM((TAIL, D), jnp.float32),
       pltpu.VMEM((G, D), jnp.float32),
       pltpu.VMEM((C,), jnp.int32),
       pltpu.VMEM_SHARED((G, D), jnp.float32)]
    + [pltpu.SemaphoreType.DMA for _ in range(2 * NBUF + 1)]
)


@functools.partial(
    pl.kernel,
    out_type=jax.ShapeDtypeStruct((2, G, D), jnp.float32),
    mesh=_mesh,
    scratch_types=_scratch,
)
def _sc_segment_sum(x_hbm, b_hbm, out_hbm, *refs):
    r_v = refs[0:NBUF]
    i_v = refs[NBUF:2 * NBUF]
    tidx_v, trows_v, lacc_v, ident_v, acc_sh = refs[2 * NBUF:2 * NBUF + 5]
    rsem = refs[2 * NBUF + 5:2 * NBUF + 5 + NBUF]
    isem = refs[2 * NBUF + 5 + NBUF:2 * NBUF + 5 + 2 * NBUF]
    dsem = refs[2 * NBUF + 5 + 2 * NBUF]

    cid = lax.axis_index("c")
    sid = lax.axis_index("s")
    wid = cid * 16 + sid
    g0 = wid * CPW

    def valid(c):
        return g0 + c < FULL_CHUNKS

    def load(c, b):
        if c >= CPW:
            return

        @pl.when(valid(c))
        def _():
            base = (g0 + c) * C
            pltpu.async_copy(b_hbm.at[pl.ds(base, C)], i_v[b], isem[b])
            pltpu.async_copy(x_hbm.at[pl.ds(base, C)], r_v[b], rsem[b])

    lane = lax.iota(jnp.int32, L)
    zv = jnp.zeros((L,), jnp.float32)

    def reduce_rows(rows_ref, ids_ref, nrows):
        def group(gi, carry):
            idv = ids_ref[pl.ds(gi * L, L)]
            first = idv[0]
            last = idv[L - 1]

            @pl.when(first == last)
            def _():
                accs = [rows_ref[gi * L, pl.ds(k * L, L)]
                        for k in range(D // L)]
                for u in range(1, L):
                    for k in range(D // L):
                        accs[k] = accs[k] + rows_ref[gi * L + u,
                                                     pl.ds(k * L, L)]
                for k in range(D // L):
                    plsc.addupdate(lacc_v.at[first, pl.ds(k * L, L)],
                                   accs[k])

            @pl.when(first != last)
            def _():
                for u in range(L):
                    rid = idv[u]
                    for k in range(D // L):
                        v = rows_ref[gi * L + u, pl.ds(k * L, L)]
                        plsc.addupdate(lacc_v.at[rid, pl.ds(k * L, L)], v)

            return carry

        lax.fori_loop(0, nrows // L, group, 0)

    def process(c, b):
        if c >= CPW:
            return

        @pl.when(valid(c))
        def _():
            base = (g0 + c) * C
            pltpu.make_async_copy(b_hbm.at[pl.ds(base, C)], i_v[b],
                                  isem[b]).wait()
            pltpu.make_async_copy(x_hbm.at[pl.ds(base, C)], r_v[b],
                                  rsem[b]).wait()
            reduce_rows(r_v[b], i_v[b], C)

    for b in range(NBUF):
        load(b, b)

    def zero_body(j, carry):
        for k in range(D // L):
            lacc_v[j, pl.ds(k * L, L)] = zv
        return carry

    lax.fori_loop(0, G, zero_body, 0)
    for k in range(C // L):
        ident_v[pl.ds(k * L, L)] = lane + (k * L)
    pltpu.sync_copy(lacc_v.at[pl.ds(0, ROWS_PER_TILE)],
                    acc_sh.at[pl.ds(sid * ROWS_PER_TILE, ROWS_PER_TILE)])
    plsc.subcore_barrier()

    def dyn_load(c, b):
        @pl.when((c < CPW) & valid(c))
        def _():
            base = (g0 + c) * C
            pltpu.async_copy(b_hbm.at[pl.ds(base, C)], i_v[b], isem[b])
            pltpu.async_copy(x_hbm.at[pl.ds(base, C)], r_v[b], rsem[b])

    def round_body(r, carry):
        for b in range(NBUF):
            c = NBUF * r + b

            @pl.when((c < CPW) & valid(c))
            def _():
                base = (g0 + c) * C
                pltpu.make_async_copy(b_hbm.at[pl.ds(base, C)], i_v[b],
                                      isem[b]).wait()
                pltpu.make_async_copy(x_hbm.at[pl.ds(base, C)], r_v[b],
                                      rsem[b]).wait()
                reduce_rows(r_v[b], i_v[b], C)

            dyn_load(c + NBUF, b)
        return carry

    lax.fori_loop(0, ROUNDS, round_body, 0)

    @pl.when(wid == NW - 1)
    def _():
        tbase = FULL_CHUNKS * C
        pltpu.sync_copy(b_hbm.at[pl.ds(tbase, TAIL)], tidx_v)
        pltpu.sync_copy(x_hbm.at[pl.ds(tbase, TAIL)], trows_v)
        reduce_rows(trows_v, tidx_v, TAIL)

    for q in range(G // C):
        pltpu.async_copy(
            lacc_v.at[pl.ds(q * C, C)],
            acc_sh.at[pl.ds(q * C, C)].at[ident_v], dsem, add=True)
    for q in range(G // C):
        pltpu.make_async_copy(
            lacc_v.at[pl.ds(q * C, C)],
            acc_sh.at[pl.ds(q * C, C)].at[ident_v], dsem).wait()

    plsc.subcore_barrier()

    pltpu.sync_copy(
        acc_sh.at[pl.ds(sid * ROWS_PER_TILE, ROWS_PER_TILE)],
        out_hbm.at[cid, pl.ds(sid * ROWS_PER_TILE, ROWS_PER_TILE)])


def _combine_body(p_ref, o_ref):
    o_ref[...] = p_ref[0] + p_ref[1]


_combine = pl.pallas_call(
    _combine_body,
    out_shape=jax.ShapeDtypeStruct((G, D), jnp.float32),
)


def kernel(input, batch, num_graphs):
    partials = _sc_segment_sum(input, batch.astype(jnp.int32))
    return _combine(partials)
